# Initial kernel scaffold; baseline (speedup 1.0000x reference)
import jax
import jax.numpy as jnp
from jax.experimental import pallas as pl


def kernel(x, A, W1, b1, W2, b2, Wout, bout):
    raise NotImplementedError("write your pallas kernel here")


# speedup vs baseline: 1.3839x; 1.3839x over previous
import jax
import jax.numpy as jnp
from jax.experimental import pallas as pl


def _elu(v):
    return jnp.where(v > 0, v, jnp.exp(jnp.minimum(v, 0.0)) - 1.0)


def _backbone_kernel(x_ref, A_ref, W1_ref, b1_ref, W2_ref, b2_ref,
                     Wout_ref, bout_ref, out_ref):
    A = A_ref[0]
    z = x_ref[0]
    W1 = W1_ref[...]

    acc = jnp.dot(z, W1[0:1], preferred_element_type=jnp.float32)
    for i in range(1, 5):
        z = jnp.dot(A, z, preferred_element_type=jnp.float32)
        acc = acc + jnp.dot(z, W1[i:i + 1], preferred_element_type=jnp.float32)
    h = _elu(acc + b1_ref[...])

    acc2 = jnp.dot(h, W2_ref[0], preferred_element_type=jnp.float32)
    y = jnp.dot(A, h, preferred_element_type=jnp.float32)
    acc2 = acc2 + jnp.dot(y, W2_ref[1], preferred_element_type=jnp.float32)
    y = jnp.dot(A, y, preferred_element_type=jnp.float32)
    acc2 = acc2 + jnp.dot(y, W2_ref[2], preferred_element_type=jnp.float32)
    h2 = _elu(acc2 + b2_ref[...])

    m = jnp.mean(h2, axis=0, keepdims=True)
    out_ref[0] = (jnp.dot(m, Wout_ref[...],
                          preferred_element_type=jnp.float32)
                  + bout_ref[...])


def kernel(x, A, W1, b1, W2, b2, Wout, bout):
    B, N, _ = x.shape
    hidden = W2.shape[-1]
    nclass = Wout.shape[-1]

    W1r = W1.reshape(W1.shape[0], hidden)
    b1r = b1.reshape(1, hidden)
    b2r = b2.reshape(1, hidden)
    boutr = bout.reshape(1, nclass)

    out = pl.pallas_call(
        _backbone_kernel,
        grid=(B,),
        in_specs=[
            pl.BlockSpec((1, N, 1), lambda b: (b, 0, 0)),
            pl.BlockSpec((1, N, N), lambda b: (b, 0, 0)),
            pl.BlockSpec(W1r.shape, lambda b: (0, 0)),
            pl.BlockSpec(b1r.shape, lambda b: (0, 0)),
            pl.BlockSpec(W2.shape, lambda b: (0, 0, 0)),
            pl.BlockSpec(b2r.shape, lambda b: (0, 0)),
            pl.BlockSpec(Wout.shape, lambda b: (0, 0)),
            pl.BlockSpec(boutr.shape, lambda b: (0, 0)),
        ],
        out_specs=pl.BlockSpec((1, 1, nclass), lambda b: (b, 0, 0)),
        out_shape=jax.ShapeDtypeStruct((B, 1, nclass), jnp.float32),
    )(x, A, W1r, b1r, W2, b2r, Wout, boutr)
    return out.reshape(B, nclass)


# speedup vs baseline: 1.3850x; 1.0008x over previous
import jax
import jax.numpy as jnp
from jax.experimental import pallas as pl


def _elu(v):
    return jnp.where(v > 0, v, jnp.exp(jnp.minimum(v, 0.0)) - 1.0)


def _backbone_kernel(x_ref, A_ref, W1_ref, b1_ref, W2_ref, b2_ref,
                     Wout_ref, bout_ref, out_ref):
    A = A_ref[0].astype(jnp.bfloat16)
    z = x_ref[0]
    W1 = W1_ref[...]

    acc = jnp.dot(z, W1[0:1], preferred_element_type=jnp.float32)
    for i in range(1, 5):
        z = jnp.dot(A, z.astype(jnp.bfloat16),
                    preferred_element_type=jnp.float32)
        acc = acc + jnp.dot(z, W1[i:i + 1], preferred_element_type=jnp.float32)
    h = _elu(acc + b1_ref[...])

    acc2 = jnp.dot(h, W2_ref[0], preferred_element_type=jnp.float32)
    y = jnp.dot(A, h.astype(jnp.bfloat16), preferred_element_type=jnp.float32)
    acc2 = acc2 + jnp.dot(y, W2_ref[1], preferred_element_type=jnp.float32)
    y = jnp.dot(A, y.astype(jnp.bfloat16), preferred_element_type=jnp.float32)
    acc2 = acc2 + jnp.dot(y, W2_ref[2], preferred_element_type=jnp.float32)
    h2 = _elu(acc2 + b2_ref[...])

    m = jnp.mean(h2, axis=0, keepdims=True)
    out_ref[0] = (jnp.dot(m, Wout_ref[...],
                          preferred_element_type=jnp.float32)
                  + bout_ref[...])


def kernel(x, A, W1, b1, W2, b2, Wout, bout):
    B, N, _ = x.shape
    hidden = W2.shape[-1]
    nclass = Wout.shape[-1]

    W1r = W1.reshape(W1.shape[0], hidden)
    b1r = b1.reshape(1, hidden)
    b2r = b2.reshape(1, hidden)
    boutr = bout.reshape(1, nclass)

    out = pl.pallas_call(
        _backbone_kernel,
        grid=(B,),
        in_specs=[
            pl.BlockSpec((1, N, 1), lambda b: (b, 0, 0)),
            pl.BlockSpec((1, N, N), lambda b: (b, 0, 0)),
            pl.BlockSpec(W1r.shape, lambda b: (0, 0)),
            pl.BlockSpec(b1r.shape, lambda b: (0, 0)),
            pl.BlockSpec(W2.shape, lambda b: (0, 0, 0)),
            pl.BlockSpec(b2r.shape, lambda b: (0, 0)),
            pl.BlockSpec(Wout.shape, lambda b: (0, 0)),
            pl.BlockSpec(boutr.shape, lambda b: (0, 0)),
        ],
        out_specs=pl.BlockSpec((1, 1, nclass), lambda b: (b, 0, 0)),
        out_shape=jax.ShapeDtypeStruct((B, 1, nclass), jnp.float32),
    )(x, A, W1r, b1r, W2, b2r, Wout, boutr)
    return out.reshape(B, nclass)


# speedup vs baseline: 1.5011x; 1.0838x over previous
import jax
import jax.numpy as jnp
from jax.experimental import pallas as pl


def _elu(v):
    return jnp.where(v > 0, v, jnp.exp(jnp.minimum(v, 0.0)) - 1.0)


def _backbone_kernel(x_ref, A_ref, W1_ref, b1_ref, W2_ref, b2_ref,
                     Wout_ref, bout_ref, out_ref):
    bpb = A_ref.shape[0]
    W1 = W1_ref[...]
    for j in range(bpb):
        A = A_ref[j].astype(jnp.bfloat16)
        z = x_ref[j]

        acc = jnp.dot(z, W1[0:1], preferred_element_type=jnp.float32)
        for i in range(1, 5):
            z = jnp.dot(A, z.astype(jnp.bfloat16),
                        preferred_element_type=jnp.float32)
            acc = acc + jnp.dot(z, W1[i:i + 1],
                                preferred_element_type=jnp.float32)
        h = _elu(acc + b1_ref[...])

        acc2 = jnp.dot(h, W2_ref[0], preferred_element_type=jnp.float32)
        y = jnp.dot(A, h.astype(jnp.bfloat16),
                    preferred_element_type=jnp.float32)
        acc2 = acc2 + jnp.dot(y, W2_ref[1], preferred_element_type=jnp.float32)
        y = jnp.dot(A, y.astype(jnp.bfloat16),
                    preferred_element_type=jnp.float32)
        acc2 = acc2 + jnp.dot(y, W2_ref[2], preferred_element_type=jnp.float32)
        h2 = _elu(acc2 + b2_ref[...])

        m = jnp.mean(h2, axis=0, keepdims=True)
        out_ref[j] = (jnp.dot(m, Wout_ref[...],
                              preferred_element_type=jnp.float32)
                      + bout_ref[...])


def kernel(x, A, W1, b1, W2, b2, Wout, bout):
    B, N, _ = x.shape
    hidden = W2.shape[-1]
    nclass = Wout.shape[-1]

    W1r = W1.reshape(W1.shape[0], hidden)
    b1r = b1.reshape(1, hidden)
    b2r = b2.reshape(1, hidden)
    boutr = bout.reshape(1, nclass)

    bpb = 2
    out = pl.pallas_call(
        _backbone_kernel,
        grid=(B // bpb,),
        in_specs=[
            pl.BlockSpec((bpb, N, 1), lambda b: (b, 0, 0)),
            pl.BlockSpec((bpb, N, N), lambda b: (b, 0, 0)),
            pl.BlockSpec(W1r.shape, lambda b: (0, 0)),
            pl.BlockSpec(b1r.shape, lambda b: (0, 0)),
            pl.BlockSpec(W2.shape, lambda b: (0, 0, 0)),
            pl.BlockSpec(b2r.shape, lambda b: (0, 0)),
            pl.BlockSpec(Wout.shape, lambda b: (0, 0)),
            pl.BlockSpec(boutr.shape, lambda b: (0, 0)),
        ],
        out_specs=pl.BlockSpec((bpb, 1, nclass), lambda b: (b, 0, 0)),
        out_shape=jax.ShapeDtypeStruct((B, 1, nclass), jnp.float32),
    )(x, A, W1r, b1r, W2, b2r, Wout, boutr)
    return out.reshape(B, nclass)


# speedup vs baseline: 2.1975x; 1.4640x over previous
import jax
import jax.numpy as jnp
from jax.experimental import pallas as pl


def _elu(v):
    return jnp.where(v > 0, v, jnp.exp(jnp.minimum(v, 0.0)) - 1.0)


def _backbone_kernel(x_ref, A_ref, W1_ref, b1_ref, W2_ref, b2_ref,
                     Wout_ref, bout_ref, out_ref):
    bpb = A_ref.shape[0]
    W1 = W1_ref[...]
    J = range(bpb)

    def dot(a, b):
        return jnp.dot(a, b, preferred_element_type=jnp.float32)

    A = [A_ref[j].astype(jnp.bfloat16) for j in J]
    z = [x_ref[j] for j in J]

    acc = [dot(z[j], W1[0:1]) for j in J]
    for i in range(1, 5):
        z = [dot(A[j], z[j].astype(jnp.bfloat16)) for j in J]
        acc = [acc[j] + dot(z[j], W1[i:i + 1]) for j in J]
    h = [_elu(acc[j] + b1_ref[...]) for j in J]

    acc2 = [dot(h[j], W2_ref[0]) for j in J]
    y = [dot(A[j], h[j].astype(jnp.bfloat16)) for j in J]
    acc2 = [acc2[j] + dot(y[j], W2_ref[1]) for j in J]
    y = [dot(A[j], y[j].astype(jnp.bfloat16)) for j in J]
    acc2 = [acc2[j] + dot(y[j], W2_ref[2]) for j in J]
    h2 = [_elu(acc2[j] + b2_ref[...]) for j in J]

    for j in J:
        m = jnp.mean(h2[j], axis=0, keepdims=True)
        out_ref[j] = dot(m, Wout_ref[...]) + bout_ref[...]


def kernel(x, A, W1, b1, W2, b2, Wout, bout):
    B, N, _ = x.shape
    hidden = W2.shape[-1]
    nclass = Wout.shape[-1]

    W1r = W1.reshape(W1.shape[0], hidden)
    b1r = b1.reshape(1, hidden)
    b2r = b2.reshape(1, hidden)
    boutr = bout.reshape(1, nclass)

    bpb = 2
    out = pl.pallas_call(
        _backbone_kernel,
        grid=(B // bpb,),
        in_specs=[
            pl.BlockSpec((bpb, N, 1), lambda b: (b, 0, 0)),
            pl.BlockSpec((bpb, N, N), lambda b: (b, 0, 0)),
            pl.BlockSpec(W1r.shape, lambda b: (0, 0)),
            pl.BlockSpec(b1r.shape, lambda b: (0, 0)),
            pl.BlockSpec(W2.shape, lambda b: (0, 0, 0)),
            pl.BlockSpec(b2r.shape, lambda b: (0, 0)),
            pl.BlockSpec(Wout.shape, lambda b: (0, 0)),
            pl.BlockSpec(boutr.shape, lambda b: (0, 0)),
        ],
        out_specs=pl.BlockSpec((bpb, 1, nclass), lambda b: (b, 0, 0)),
        out_shape=jax.ShapeDtypeStruct((B, 1, nclass), jnp.float32),
    )(x, A, W1r, b1r, W2, b2r, Wout, boutr)
    return out.reshape(B, nclass)
